# baseline (speedup 1.0000x reference)
import functools

import jax
import jax.numpy as jnp
from jax import lax
from jax.experimental import pallas as pl
from jax.experimental.pallas import tpu as pltpu
from jax.experimental.pallas import tpu_sc as plsc

N_IN = 16777216
N_OUT = 2 * N_IN
NUM_CORES = 2
NUM_SUBCORES = 16
NW = NUM_CORES * NUM_SUBCORES
S = N_IN // NW
C = 32768
NCHUNK = S // C
NB = 3

_mesh = plsc.VectorSubcoreMesh(
    core_axis_name="c", subcore_axis_name="s", num_cores=NUM_CORES)


@functools.partial(
    pl.kernel,
    mesh=_mesh,
    out_type=jax.ShapeDtypeStruct((N_OUT,), jnp.float32),
    scratch_types=[
        pltpu.VMEM((C,), jnp.float32),
        pltpu.VMEM((C,), jnp.float32),
        pltpu.VMEM((C,), jnp.float32),
        pltpu.SemaphoreType.DMA,
        pltpu.SemaphoreType.DMA,
        pltpu.SemaphoreType.DMA,
        pltpu.SemaphoreType.DMA,
        pltpu.SemaphoreType.DMA,
        pltpu.SemaphoreType.DMA,
    ],
)
def _copy_lower(psi_hbm, out_hbm, buf0, buf1, buf2,
                sem_r0, sem_r1, sem_r2, sem_w0, sem_w1, sem_w2):
    wid = lax.axis_index("s") * NUM_CORES + lax.axis_index("c")
    base = wid * S
    bufs = (buf0, buf1, buf2)
    sem_r = (sem_r0, sem_r1, sem_r2)
    sem_w = (sem_w0, sem_w1, sem_w2)

    def read(i):
        b = i % NB
        return pltpu.async_copy(
            psi_hbm.at[pl.ds(base + i * C, C)], bufs[b], sem_r[b])

    def write(i):
        b = i % NB
        return pltpu.async_copy(
            bufs[b], out_hbm.at[pl.ds(base + i * C, C)], sem_w[b])

    rd, wr = {}, {}
    for i in range(NB):
        rd[i] = read(i)
    for j in range(NB - 1):
        rd[j].wait()
        wr[j] = write(j)
    for i in range(NB, NCHUNK + 1):
        if i < NCHUNK:
            wr[i - NB].wait()
            rd[i] = read(i)
        j = i - 1
        rd[j].wait()
        wr[j] = write(j)
    for j in range(NCHUNK - NB, NCHUNK):
        wr[j].wait()


ZBLK = 524288
NZBLK = N_IN // ZBLK


def _zero_upper_body(full_ref, out_ref):
    out_ref[...] = jnp.zeros((ZBLK,), jnp.float32)


_zero_upper = pl.pallas_call(
    _zero_upper_body,
    grid=(NZBLK,),
    in_specs=[pl.BlockSpec(memory_space=pl.ANY)],
    out_specs=pl.BlockSpec((ZBLK,), lambda i: (NZBLK + i,)),
    out_shape=jax.ShapeDtypeStruct((N_OUT,), jnp.float32),
    input_output_aliases={0: 0},
)


def kernel(psi):
    return _zero_upper(_copy_lower(psi))


# speedup vs baseline: 1.0537x; 1.0537x over previous
import functools

import jax
import jax.numpy as jnp
from jax import lax
from jax.experimental import pallas as pl
from jax.experimental.pallas import tpu as pltpu
from jax.experimental.pallas import tpu_sc as plsc

N_IN = 16777216
N_OUT = 2 * N_IN
NUM_CORES = 2
NUM_SUBCORES = 16
NW = NUM_CORES * NUM_SUBCORES
S = N_IN // NW
C = 49152
CHUNKS = [(i * C, C) for i in range(10)] + [(10 * C, S - 10 * C)]
NCHUNK = len(CHUNKS)
NB = 2

_mesh = plsc.VectorSubcoreMesh(
    core_axis_name="c", subcore_axis_name="s", num_cores=NUM_CORES)


@functools.partial(
    pl.kernel,
    mesh=_mesh,
    out_type=jax.ShapeDtypeStruct((N_OUT,), jnp.float32),
    scratch_types=[
        pltpu.VMEM((C,), jnp.float32),
        pltpu.VMEM((C,), jnp.float32),
        pltpu.SemaphoreType.DMA,
        pltpu.SemaphoreType.DMA,
        pltpu.SemaphoreType.DMA,
        pltpu.SemaphoreType.DMA,
    ],
)
def _copy_lower(psi_hbm, out_hbm, buf0, buf1,
                sem_r0, sem_r1, sem_w0, sem_w1):
    wid = lax.axis_index("s") * NUM_CORES + lax.axis_index("c")
    base = wid * S
    bufs = (buf0, buf1)
    sem_r = (sem_r0, sem_r1)
    sem_w = (sem_w0, sem_w1)

    def staged(i):
        b = i % NB
        off, sz = CHUNKS[i]
        return bufs[b] if sz == C else bufs[b].at[pl.ds(0, sz)]

    def read(i):
        off, sz = CHUNKS[i]
        return pltpu.async_copy(
            psi_hbm.at[pl.ds(base + off, sz)], staged(i), sem_r[i % NB])

    def write(i):
        off, sz = CHUNKS[i]
        return pltpu.async_copy(
            staged(i), out_hbm.at[pl.ds(base + off, sz)], sem_w[i % NB])

    rd, wr = {}, {}
    for i in range(NB):
        rd[i] = read(i)
    for j in range(NB - 1):
        rd[j].wait()
        wr[j] = write(j)
    for i in range(NB, NCHUNK + 1):
        if i < NCHUNK:
            wr[i - NB].wait()
            rd[i] = read(i)
        j = i - 1
        rd[j].wait()
        wr[j] = write(j)
    for j in range(NCHUNK - NB, NCHUNK):
        wr[j].wait()


ZBLK = 1048576
NZBLK = N_IN // ZBLK


def _zero_upper_body(full_ref, out_ref):
    out_ref[...] = jnp.zeros((ZBLK,), jnp.float32)


_zero_upper = pl.pallas_call(
    _zero_upper_body,
    grid=(NZBLK,),
    in_specs=[pl.BlockSpec(memory_space=pl.ANY)],
    out_specs=pl.BlockSpec((ZBLK,), lambda i: (NZBLK + i,)),
    out_shape=jax.ShapeDtypeStruct((N_OUT,), jnp.float32),
    input_output_aliases={0: 0},
)


def kernel(psi):
    return _zero_upper(_copy_lower(psi))


# speedup vs baseline: 1.0609x; 1.0068x over previous
import functools

import jax
import jax.numpy as jnp
from jax import lax
from jax.experimental import pallas as pl
from jax.experimental.pallas import tpu as pltpu
from jax.experimental.pallas import tpu_sc as plsc

N_IN = 16777216
N_OUT = 2 * N_IN
NUM_CORES = 2
NUM_SUBCORES = 16
NW = NUM_CORES * NUM_SUBCORES
S = N_IN // NW
C = 65528
CHUNKS = [(i * C, C) for i in range(8)] + [(8 * C, S - 8 * C)]
NCHUNK = len(CHUNKS)
NB = 2

_mesh = plsc.VectorSubcoreMesh(
    core_axis_name="c", subcore_axis_name="s", num_cores=NUM_CORES)


@functools.partial(
    pl.kernel,
    mesh=_mesh,
    out_type=jax.ShapeDtypeStruct((N_OUT,), jnp.float32),
    scratch_types=[
        pltpu.VMEM((C,), jnp.float32),
        pltpu.VMEM((C,), jnp.float32),
        pltpu.SemaphoreType.DMA,
        pltpu.SemaphoreType.DMA,
        pltpu.SemaphoreType.DMA,
        pltpu.SemaphoreType.DMA,
    ],
)
def _copy_lower(psi_hbm, out_hbm, buf0, buf1,
                sem_r0, sem_r1, sem_w0, sem_w1):
    wid = lax.axis_index("s") * NUM_CORES + lax.axis_index("c")
    base = wid * S
    bufs = (buf0, buf1)
    sem_r = (sem_r0, sem_r1)
    sem_w = (sem_w0, sem_w1)

    def staged(i):
        b = i % NB
        off, sz = CHUNKS[i]
        return bufs[b] if sz == C else bufs[b].at[pl.ds(0, sz)]

    def read(i):
        off, sz = CHUNKS[i]
        return pltpu.async_copy(
            psi_hbm.at[pl.ds(base + off, sz)], staged(i), sem_r[i % NB])

    def write(i):
        off, sz = CHUNKS[i]
        return pltpu.async_copy(
            staged(i), out_hbm.at[pl.ds(base + off, sz)], sem_w[i % NB])

    rd, wr = {}, {}
    for i in range(NB):
        rd[i] = read(i)
    for j in range(NB - 1):
        rd[j].wait()
        wr[j] = write(j)
    for i in range(NB, NCHUNK + 1):
        if i < NCHUNK:
            wr[i - NB].wait()
            rd[i] = read(i)
        j = i - 1
        rd[j].wait()
        wr[j] = write(j)
    for j in range(NCHUNK - NB, NCHUNK):
        wr[j].wait()


ZBLK = 1048576
NZBLK = N_IN // ZBLK


def _zero_upper_body(full_ref, out_ref):
    out_ref[...] = jnp.zeros((ZBLK,), jnp.float32)


_zero_upper = pl.pallas_call(
    _zero_upper_body,
    grid=(NZBLK,),
    in_specs=[pl.BlockSpec(memory_space=pl.ANY)],
    out_specs=pl.BlockSpec((ZBLK,), lambda i: (NZBLK + i,)),
    out_shape=jax.ShapeDtypeStruct((N_OUT,), jnp.float32),
    input_output_aliases={0: 0},
)


def kernel(psi):
    return _zero_upper(_copy_lower(psi))
